# baseline (speedup 1.0000x reference)
import math

import jax
import jax.numpy as jnp
from jax import lax
from jax.experimental import pallas as pl
from jax.experimental.pallas import tpu as pltpu

L = 2048
D = 1024
H = 16
DH = 64
BL = 256
NBLK = L // BL
PAD = 1024
OFFSETS = tuple(2 ** j for j in range(11))
SCALE = 1.0 / math.sqrt(DH)


def _selectors(dtype):
    sel = (lax.broadcasted_iota(jnp.int32, (D, H), 0) // DH
           == lax.broadcasted_iota(jnp.int32, (D, H), 1)).astype(dtype)
    selT = (lax.broadcasted_iota(jnp.int32, (H, D), 1) // DH
            == lax.broadcasted_iota(jnp.int32, (H, D), 0)).astype(dtype)
    return sel, selT


def _fused_kernel(x_ref, wq_ref, wk_ref, wv_ref, wo_ref,
                  bq_ref, bk_ref, bv_ref, bo_ref, o_ref,
                  kp_s, vp_s, u_s):
    f32 = jnp.float32
    pid = pl.program_id(0)

    @pl.when(pid == 0)
    def _vproj():
        vb = (jnp.dot(x_ref[...], wv_ref[...], preferred_element_type=f32)
              + bv_ref[...])
        vp_s[pl.ds(PAD, L), :] = vb
        kp_s[pl.ds(0, PAD), :] = jnp.zeros((PAD, D), f32)
        vp_s[pl.ds(0, PAD), :] = jnp.zeros((PAD, D), f32)
        sv = jnp.sum(vb, axis=0, keepdims=True)
        _, selT = _selectors(f32)
        u_s[...] = jnp.dot(selT * sv, wo_ref[...], preferred_element_type=f32)

    @pl.when(pid > 0)
    def _block():
        b = pid - 1
        i0 = b * BL
        xb = x_ref[pl.ds(i0, BL), :]
        q = (jnp.dot(xb, wq_ref[...], preferred_element_type=f32)
             + bq_ref[...])
        kb = (jnp.dot(xb, wk_ref[...], preferred_element_type=f32)
              + bk_ref[...])
        kp_s[pl.ds(PAD + i0, BL), :] = kb

        sel, selT = _selectors(f32)

        kw = kp_s[pl.ds(i0 + PAD - 8, BL + 8), :]
        vw = vp_s[pl.ds(i0 + PAD - 8, BL + 8), :]

        kd = kw[8:8 + BL, :]
        vd = vw[8:8 + BL, :]
        s = jnp.dot(q * kd, sel, preferred_element_type=f32) * SCALE
        w = jnp.exp(s) - 1.0
        z = w + float(L)
        acc = jnp.dot(w, selT, preferred_element_type=f32) * vd

        for d in OFFSETS:
            if d % 8 == 0:
                ks = kp_s[pl.ds(i0 + PAD - d, BL), :]
                vs = vp_s[pl.ds(i0 + PAD - d, BL), :]
            else:
                ks = kw[8 - d:8 - d + BL, :]
                vs = vw[8 - d:8 - d + BL, :]
            s = jnp.dot(q * ks, sel, preferred_element_type=f32) * SCALE
            w = jnp.exp(s) - 1.0
            z += w
            acc += jnp.dot(w, selT, preferred_element_type=f32) * vs

        zinv = 1.0 / z
        att_main = acc * jnp.dot(zinv, selT, preferred_element_type=f32)
        o_ref[...] = (
            jnp.dot(att_main, wo_ref[...], preferred_element_type=f32)
            + jnp.dot(zinv, u_s[...], preferred_element_type=f32)
            + bo_ref[...])


@jax.jit
def kernel(x, Wq, bq, Wk, bk, Wv, bv, Wo, bo):
    x2 = x.reshape(L, D)
    bq2 = bq.reshape(1, D)
    bk2 = bk.reshape(1, D)
    bv2 = bv.reshape(1, D)
    bo2 = bo.reshape(1, D)

    full = lambda shape: pl.BlockSpec(shape, lambda i: (0, 0))

    out = pl.pallas_call(
        _fused_kernel,
        grid=(1 + NBLK,),
        in_specs=[
            full((L, D)),
            full((D, D)), full((D, D)), full((D, D)), full((D, D)),
            full((1, D)), full((1, D)), full((1, D)), full((1, D)),
        ],
        out_specs=pl.BlockSpec((BL, D), lambda i: (jnp.maximum(i - 1, 0), 0)),
        out_shape=jax.ShapeDtypeStruct((L, D), jnp.float32),
        scratch_shapes=[
            pltpu.VMEM((PAD + L, D), jnp.float32),
            pltpu.VMEM((PAD + L, D), jnp.float32),
            pltpu.VMEM((H, D), jnp.float32),
        ],
        compiler_params=pltpu.CompilerParams(
            dimension_semantics=("arbitrary",),
            vmem_limit_bytes=100 * 1024 * 1024),
    )(x2, Wq, Wk, Wv, Wo, bq2, bk2, bv2, bo2)

    return out.reshape(1, L, D)


# speedup vs baseline: 1.0436x; 1.0436x over previous
import math

import jax
import jax.numpy as jnp
from jax import lax
from jax.experimental import pallas as pl
from jax.experimental.pallas import tpu as pltpu

L = 2048
D = 1024
H = 16
DH = 64
BL = 512
NBLK = L // BL
PAD = 1024
OFFSETS = tuple(2 ** j for j in range(11))
SCALE = 1.0 / math.sqrt(DH)


def _selectors(dtype):
    sel = (lax.broadcasted_iota(jnp.int32, (D, H), 0) // DH
           == lax.broadcasted_iota(jnp.int32, (D, H), 1)).astype(dtype)
    selT = (lax.broadcasted_iota(jnp.int32, (H, D), 1) // DH
            == lax.broadcasted_iota(jnp.int32, (H, D), 0)).astype(dtype)
    return sel, selT


def _fused_kernel(x_ref, wq_ref, wk_ref, wv_ref, wo_ref,
                  bq_ref, bk_ref, bv_ref, bo_ref, o_ref,
                  kp_s, vp_s, u_s):
    f32 = jnp.float32
    pid = pl.program_id(0)

    @pl.when(pid == 0)
    def _vproj():
        vb = (jnp.dot(x_ref[...], wv_ref[...], preferred_element_type=f32)
              + bv_ref[...])
        vp_s[pl.ds(PAD, L), :] = vb
        kp_s[pl.ds(0, PAD), :] = jnp.zeros((PAD, D), f32)
        vp_s[pl.ds(0, PAD), :] = jnp.zeros((PAD, D), f32)
        sv = jnp.sum(vb, axis=0, keepdims=True)
        _, selT = _selectors(f32)
        u_s[...] = jnp.dot(selT * sv, wo_ref[...], preferred_element_type=f32)

    @pl.when(pid > 0)
    def _block():
        b = pid - 1
        i0 = b * BL
        xb = x_ref[pl.ds(i0, BL), :]
        q = (jnp.dot(xb, wq_ref[...], preferred_element_type=f32)
             + bq_ref[...])
        kb = (jnp.dot(xb, wk_ref[...], preferred_element_type=f32)
              + bk_ref[...])
        kp_s[pl.ds(PAD + i0, BL), :] = kb

        sel, selT = _selectors(f32)

        kw = kp_s[pl.ds(i0 + PAD - 8, BL + 8), :]
        vw = vp_s[pl.ds(i0 + PAD - 8, BL + 8), :]

        kd = kw[8:8 + BL, :]
        vd = vw[8:8 + BL, :]
        s = jnp.dot(q * kd, sel, preferred_element_type=f32) * SCALE
        w = jnp.exp(s) - 1.0
        z = w + float(L)
        acc = jnp.dot(w, selT, preferred_element_type=f32) * vd

        for d in OFFSETS:
            if d % 8 == 0:
                ks = kp_s[pl.ds(i0 + PAD - d, BL), :]
                vs = vp_s[pl.ds(i0 + PAD - d, BL), :]
            else:
                ks = kw[8 - d:8 - d + BL, :]
                vs = vw[8 - d:8 - d + BL, :]
            s = jnp.dot(q * ks, sel, preferred_element_type=f32) * SCALE
            w = jnp.exp(s) - 1.0
            z += w
            acc += jnp.dot(w, selT, preferred_element_type=f32) * vs

        zinv = 1.0 / z
        att_main = acc * jnp.dot(zinv, selT, preferred_element_type=f32)
        o_ref[...] = (
            jnp.dot(att_main, wo_ref[...], preferred_element_type=f32)
            + jnp.dot(zinv, u_s[...], preferred_element_type=f32)
            + bo_ref[...])


@jax.jit
def kernel(x, Wq, bq, Wk, bk, Wv, bv, Wo, bo):
    x2 = x.reshape(L, D)
    bq2 = bq.reshape(1, D)
    bk2 = bk.reshape(1, D)
    bv2 = bv.reshape(1, D)
    bo2 = bo.reshape(1, D)

    full = lambda shape: pl.BlockSpec(shape, lambda i: (0, 0))

    out = pl.pallas_call(
        _fused_kernel,
        grid=(1 + NBLK,),
        in_specs=[
            full((L, D)),
            full((D, D)), full((D, D)), full((D, D)), full((D, D)),
            full((1, D)), full((1, D)), full((1, D)), full((1, D)),
        ],
        out_specs=pl.BlockSpec((BL, D), lambda i: (jnp.maximum(i - 1, 0), 0)),
        out_shape=jax.ShapeDtypeStruct((L, D), jnp.float32),
        scratch_shapes=[
            pltpu.VMEM((PAD + L, D), jnp.float32),
            pltpu.VMEM((PAD + L, D), jnp.float32),
            pltpu.VMEM((H, D), jnp.float32),
        ],
        compiler_params=pltpu.CompilerParams(
            dimension_semantics=("arbitrary",),
            vmem_limit_bytes=100 * 1024 * 1024),
    )(x2, Wq, Wk, Wv, Wo, bq2, bk2, bv2, bo2)

    return out.reshape(1, L, D)
